# baseline (speedup 1.0000x reference)
import functools

import jax
import jax.numpy as jnp
from jax import lax
from jax.experimental import pallas as pl
from jax.experimental.pallas import tpu as pltpu
from jax.experimental.pallas import tpu_sc as plsc

N_FIELDS_K = 26
EMBED_DIM_K = 32
BATCH_K = 16384

NUM_WORKERS = 32
RPW = BATCH_K // NUM_WORKERS
NLANE = 16


def _sc_body(table_h, x_h, bias_h, out_h,
             xblk, bias_v, idx0, idx1, stg0, stg1, trb0, trb1,
             sg0, sg1, ss0, ss1):
    wid = lax.axis_index("s") * 2 + lax.axis_index("c")
    b0 = wid * RPW

    pltpu.sync_copy(x_h.at[pl.ds(b0, RPW)], xblk)
    pltpu.sync_copy(bias_h, bias_v)

    iota = lax.iota(jnp.int32, NLANE)
    vh0 = (iota // 8) * 4096 + (iota % 8) * 128
    vh1 = vh0 + 2 * 4096

    def build_idx(f, idxb):
        def per_k(k, _):
            for g in range(8):
                rows = k * 128 + g * NLANE + iota
                cols = jnp.broadcast_to(f, (NLANE,))
                v = plsc.load_gather(xblk, [rows, cols])
                idxb[k, pl.ds(g * NLANE, NLANE)] = v
            return 0
        lax.fori_loop(0, 4, per_k, 0)

    def fire(idxb, stg, sem):
        for k in range(4):
            pltpu.async_copy(
                table_h.at[idxb.at[k]],
                stg.at[pl.ds(k * 128, 128)],
                sem,
            )

    def drain(stg, sem):
        pltpu.make_async_copy(table_h.at[pl.ds(0, RPW)], stg, sem).wait()

    def compute(f, stg, trb):
        bias_lo = bias_v[f, pl.ds(0, NLANE)]
        bias_hi = bias_v[f, pl.ds(NLANE, NLANE)]

        def per_i(i, _):
            for tbl in range(4):
                for j in range(4):
                    l = tbl * 128 + i * 4 + j
                    pos = jnp.broadcast_to(tbl * 1024 + i * 4 + j, (NLANE,))
                    lo = stg[l, pl.ds(0, NLANE)] + bias_lo
                    hi = stg[l, pl.ds(NLANE, NLANE)] + bias_hi
                    plsc.store_scatter(trb, [vh0 + pos], lo)
                    plsc.store_scatter(trb, [vh1 + pos], hi)
            return 0
        lax.fori_loop(0, 32, per_i, 0)

    def scatter_out(f, trb, sem):
        for ts in range(4):
            pltpu.async_copy(
                trb.at[pl.ds(ts * 4096, 4096)],
                out_h.at[f, pl.ds(ts * 131072 + wid * 4096, 4096)],
                sem,
            )

    def wait_scatter(f, trb, sem):
        for ts in range(4):
            pltpu.make_async_copy(
                trb.at[pl.ds(ts * 4096, 4096)],
                out_h.at[f, pl.ds(ts * 131072 + wid * 4096, 4096)],
                sem,
            ).wait()

    build_idx(0, idx0)
    fire(idx0, stg0, sg0)

    def pair_step(p, _):
        f0 = 2 * p
        f1 = f0 + 1

        @pl.when(p > 0)
        def _():
            wait_scatter(f1 - 2, trb1, ss1)
        build_idx(f1, idx1)
        fire(idx1, stg1, sg1)

        drain(stg0, sg0)
        compute(f0, stg0, trb0)
        scatter_out(f0, trb0, ss0)

        @pl.when(p < N_FIELDS_K // 2 - 1)
        def _():
            wait_scatter(f0, trb0, ss0)
            build_idx(f0 + 2, idx0)
            fire(idx0, stg0, sg0)

        drain(stg1, sg1)
        compute(f1, stg1, trb1)
        scatter_out(f1, trb1, ss1)
        return 0

    lax.fori_loop(0, N_FIELDS_K // 2, pair_step, 0)
    wait_scatter(N_FIELDS_K - 2, trb0, ss0)
    wait_scatter(N_FIELDS_K - 1, trb1, ss1)


@jax.jit
def kernel(X, table, bias):
    mesh = plsc.VectorSubcoreMesh(core_axis_name="c", subcore_axis_name="s")
    run = functools.partial(
        pl.kernel,
        mesh=mesh,
        out_type=jax.ShapeDtypeStruct((N_FIELDS_K, 4 * 131072), jnp.float32),
        scratch_types=[
            pltpu.VMEM((RPW, N_FIELDS_K), jnp.int32),
            pltpu.VMEM((N_FIELDS_K, EMBED_DIM_K), jnp.float32),
            pltpu.VMEM((4, 128), jnp.int32),
            pltpu.VMEM((4, 128), jnp.int32),
            pltpu.VMEM((RPW, EMBED_DIM_K), jnp.float32),
            pltpu.VMEM((RPW, EMBED_DIM_K), jnp.float32),
            pltpu.VMEM((4 * 4096,), jnp.float32),
            pltpu.VMEM((4 * 4096,), jnp.float32),
            pltpu.SemaphoreType.DMA,
            pltpu.SemaphoreType.DMA,
            pltpu.SemaphoreType.DMA,
            pltpu.SemaphoreType.DMA,
        ],
        compiler_params=pltpu.CompilerParams(use_tc_tiling_on_sc=False,
                                             needs_layout_passes=False),
    )(_sc_body)
    out2 = run(table, X, bias)
    out5 = out2.reshape(N_FIELDS_K, 4, BATCH_K // 128, 8, 128)
    return out5.transpose(2, 4, 0, 1, 3).reshape(BATCH_K, N_FIELDS_K,
                                                 EMBED_DIM_K)


# speedup vs baseline: 1.3128x; 1.3128x over previous
import functools

import jax
import jax.numpy as jnp
from jax import lax
from jax.experimental import pallas as pl
from jax.experimental.pallas import tpu as pltpu
from jax.experimental.pallas import tpu_sc as plsc

N_FIELDS_K = 26
EMBED_DIM_K = 32
BATCH_K = 16384

NUM_WORKERS = 32
RPW = BATCH_K // NUM_WORKERS
NLANE = 16


def _sc_body(table_h, x_h, bias_h, out_h,
             xblk, bias_v, idx0, idx1, stg0, stg1, trb0, trb1,
             sg0, sg1, ss0, ss1):
    wid = lax.axis_index("s") * 2 + lax.axis_index("c")
    b0 = wid * RPW

    pltpu.sync_copy(x_h.at[pl.ds(b0, RPW)], xblk)
    pltpu.sync_copy(bias_h, bias_v)

    iota = lax.iota(jnp.int32, NLANE)
    d_lo = iota
    d_hi = iota + NLANE

    def build_idx(f, idxb):
        def per_k(k, _):
            for g in range(8):
                rows = k * 128 + g * NLANE + iota
                cols = jnp.broadcast_to(f, (NLANE,))
                v = plsc.load_gather(xblk, [rows, cols])
                idxb[k, pl.ds(g * NLANE, NLANE)] = v
            return 0
        lax.fori_loop(0, 4, per_k, 0)

    def fire(idxb, stg, sem):
        for k in range(4):
            pltpu.async_copy(
                table_h.at[idxb.at[k]],
                stg.at[pl.ds(k * 128, 128)],
                sem,
            )

    def drain(stg, sem):
        pltpu.make_async_copy(table_h.at[pl.ds(0, RPW)], stg, sem).wait()

    def compute(f, stg, trb):
        bias_lo = bias_v[f, pl.ds(0, NLANE)]
        bias_hi = bias_v[f, pl.ds(NLANE, NLANE)]

        def per_i(i, _):
            for tbl in range(4):
                for j in range(4):
                    l = tbl * 128 + i * 4 + j
                    col = jnp.broadcast_to(tbl * 128 + i * 4 + j, (NLANE,))
                    lo = stg[l, pl.ds(0, NLANE)] + bias_lo
                    hi = stg[l, pl.ds(NLANE, NLANE)] + bias_hi
                    plsc.store_scatter(trb, [d_lo, col], lo)
                    plsc.store_scatter(trb, [d_hi, col], hi)
            return 0
        lax.fori_loop(0, 32, per_i, 0)

    def scatter_out(f, trb, sem):
        for ts in range(4):
            for tbl in range(4):
                pltpu.async_copy(
                    trb.at[pl.ds(ts * 8, 8), pl.ds(tbl * 128, 128)],
                    out_h.at[f, ts, wid * 4 + tbl],
                    sem,
                )

    def wait_scatter(f, trb, sem):
        for ts in range(4):
            for tbl in range(4):
                pltpu.make_async_copy(
                    trb.at[pl.ds(ts * 8, 8), pl.ds(tbl * 128, 128)],
                    out_h.at[f, ts, wid * 4 + tbl],
                    sem,
                ).wait()

    build_idx(0, idx0)
    fire(idx0, stg0, sg0)

    def pair_step(p, _):
        f0 = 2 * p
        f1 = f0 + 1

        @pl.when(p > 0)
        def _():
            wait_scatter(f1 - 2, trb1, ss1)
        build_idx(f1, idx1)
        fire(idx1, stg1, sg1)

        drain(stg0, sg0)
        compute(f0, stg0, trb0)
        scatter_out(f0, trb0, ss0)

        @pl.when(p < N_FIELDS_K // 2 - 1)
        def _():
            wait_scatter(f0, trb0, ss0)
            build_idx(f0 + 2, idx0)
            fire(idx0, stg0, sg0)

        drain(stg1, sg1)
        compute(f1, stg1, trb1)
        scatter_out(f1, trb1, ss1)
        return 0

    lax.fori_loop(0, N_FIELDS_K // 2, pair_step, 0)
    wait_scatter(N_FIELDS_K - 2, trb0, ss0)
    wait_scatter(N_FIELDS_K - 1, trb1, ss1)


@jax.jit
def kernel(X, table, bias):
    mesh = plsc.VectorSubcoreMesh(core_axis_name="c", subcore_axis_name="s")
    run = functools.partial(
        pl.kernel,
        mesh=mesh,
        out_type=jax.ShapeDtypeStruct(
            (N_FIELDS_K, 4, BATCH_K // 128, 8, 128), jnp.float32),
        scratch_types=[
            pltpu.VMEM((RPW, N_FIELDS_K), jnp.int32),
            pltpu.VMEM((N_FIELDS_K, EMBED_DIM_K), jnp.float32),
            pltpu.VMEM((4, 128), jnp.int32),
            pltpu.VMEM((4, 128), jnp.int32),
            pltpu.VMEM((RPW, EMBED_DIM_K), jnp.float32),
            pltpu.VMEM((RPW, EMBED_DIM_K), jnp.float32),
            pltpu.VMEM((EMBED_DIM_K, 513), jnp.float32),
            pltpu.VMEM((EMBED_DIM_K, 513), jnp.float32),
            pltpu.SemaphoreType.DMA,
            pltpu.SemaphoreType.DMA,
            pltpu.SemaphoreType.DMA,
            pltpu.SemaphoreType.DMA,
        ],
        compiler_params=pltpu.CompilerParams(use_tc_tiling_on_sc=False,
                                             needs_layout_passes=False),
    )(_sc_body)
    out5 = run(table, X, bias)
    return out5.transpose(2, 4, 0, 1, 3).reshape(BATCH_K, N_FIELDS_K,
                                                 EMBED_DIM_K)
